# baseline (speedup 1.0000x reference)
import jax
import jax.numpy as jnp
from jax import lax
from jax.experimental import pallas as pl
from jax.experimental.pallas import tpu as pltpu

N_ROWS = 8192
K = 8192
D = 32
TILE = 256
GRID = N_ROWS // TILE
CHUNK = 2048


def _vq_tile_kernel(x_ref, x2_ref, e2_ref, embed_ref, embed_t_ref,
                    idx_ref, counts_ref, quant_ref, counts_acc):
    i = pl.program_id(0)

    @pl.when(i == 0)
    def _():
        counts_acc[...] = jnp.zeros_like(counts_acc)

    x = x_ref[...]
    xe = jnp.dot(x, embed_ref[...], preferred_element_type=jnp.float32)
    scores = (x2_ref[...] - 2.0 * xe) + e2_ref[...]
    big = jnp.int32(2 ** 30)
    v = jnp.full((TILE, 1), jnp.inf, jnp.float32)
    idx = jnp.zeros((TILE, 1), jnp.int32)
    for c in range(K // CHUNK):
        chunk = scores[:, c * CHUNK:(c + 1) * CHUNK]
        mc = jnp.min(chunk, axis=1, keepdims=True)
        iota_c = lax.broadcasted_iota(jnp.int32, (TILE, CHUNK), 1) + c * CHUNK
        ic = jnp.min(jnp.where(chunk == mc, iota_c, big), axis=1, keepdims=True)
        win = mc < v
        idx = jnp.where(win, ic, idx)
        v = jnp.where(win, mc, v)
        u = lax.bitcast_convert_type(v, jnp.uint32)
        u = (u + jnp.uint32(0x8000)) & jnp.uint32(0xFFFF0000)
        v = lax.bitcast_convert_type(u, jnp.float32)
    idx_ref[...] = idx
    iota = lax.broadcasted_iota(jnp.int32, (TILE, K), 1)
    onehot = (iota == idx).astype(jnp.float32)
    counts_acc[...] += jnp.sum(onehot, axis=0, keepdims=True)
    quant_ref[...] = jnp.dot(onehot, embed_t_ref[...],
                             preferred_element_type=jnp.float32,
                             precision=lax.Precision.HIGHEST)

    @pl.when(i == GRID - 1)
    def _():
        counts_ref[...] = counts_acc[...]


def _finish_kernel(x_ref, q_ref, counts_ref, qst_ref, diff_ref, ppl_ref):
    x = x_ref[...]
    q = q_ref[...]
    d = q - x
    qst_ref[...] = x + d
    diff_ref[...] = (jnp.sum(d * d) / (N_ROWS * D)).reshape(1, 1)
    p = counts_ref[...] * (1.0 / N_ROWS)
    ent = jnp.sum(p * jnp.log(p + 1e-10))
    ppl_ref[...] = jnp.exp(-ent).reshape(1, 1)


def kernel(x, embed):
    x_flat = x.reshape(-1, D)
    x2 = (x_flat ** 2).sum(axis=1, keepdims=True)
    e2 = (embed ** 2).sum(axis=0, keepdims=True)
    embed_t = embed.T

    idx2, counts, quant = pl.pallas_call(
        _vq_tile_kernel,
        grid=(GRID,),
        in_specs=[
            pl.BlockSpec((TILE, D), lambda i: (i, 0)),
            pl.BlockSpec((TILE, 1), lambda i: (i, 0)),
            pl.BlockSpec((1, K), lambda i: (0, 0)),
            pl.BlockSpec((D, K), lambda i: (0, 0)),
            pl.BlockSpec((K, D), lambda i: (0, 0)),
        ],
        out_specs=[
            pl.BlockSpec((TILE, 1), lambda i: (i, 0)),
            pl.BlockSpec((1, K), lambda i: (0, 0)),
            pl.BlockSpec((TILE, D), lambda i: (i, 0)),
        ],
        out_shape=[
            jax.ShapeDtypeStruct((N_ROWS, 1), jnp.int32),
            jax.ShapeDtypeStruct((1, K), jnp.float32),
            jax.ShapeDtypeStruct((N_ROWS, D), jnp.float32),
        ],
        scratch_shapes=[pltpu.VMEM((1, K), jnp.float32)],
    )(x_flat, x2, e2, embed, embed_t)

    qst, diff, ppl = pl.pallas_call(
        _finish_kernel,
        out_shape=[
            jax.ShapeDtypeStruct((N_ROWS, D), jnp.float32),
            jax.ShapeDtypeStruct((1, 1), jnp.float32),
            jax.ShapeDtypeStruct((1, 1), jnp.float32),
        ],
    )(x_flat, quant, counts)

    quantized_st = qst.reshape(x.shape)
    encoding_inds = idx2.reshape(x.shape[:-1])
    return (quantized_st, diff.reshape(()), encoding_inds, ppl.reshape(()))


# speedup vs baseline: 2.0596x; 2.0596x over previous
import functools

import jax
import jax.numpy as jnp
from jax import lax
from jax.experimental import pallas as pl
from jax.experimental.pallas import tpu as pltpu
from jax.experimental.pallas import tpu_sc as plsc

N_ROWS = 8192
K = 8192
D = 32
TILE = 256
GRID = N_ROWS // TILE
CHUNK = 2048

NW = 32
BPW = N_ROWS // NW


def _vq_tile_kernel(x_ref, x2_ref, e2_ref, embed_ref,
                    idx_ref, counts_ref, counts_acc):
    i = pl.program_id(0)

    @pl.when(i == 0)
    def _():
        counts_acc[...] = jnp.zeros_like(counts_acc)

    x = x_ref[...]
    xe = jnp.dot(x, embed_ref[...], preferred_element_type=jnp.float32)
    scores = (x2_ref[...] - 2.0 * xe) + e2_ref[...]
    big = jnp.int32(2 ** 30)
    v = jnp.full((TILE, 1), jnp.inf, jnp.float32)
    idx = jnp.zeros((TILE, 1), jnp.int32)
    for c in range(K // CHUNK):
        chunk = scores[:, c * CHUNK:(c + 1) * CHUNK]
        mc = jnp.min(chunk, axis=1, keepdims=True)
        iota_c = lax.broadcasted_iota(jnp.int32, (TILE, CHUNK), 1) + c * CHUNK
        ic = jnp.min(jnp.where(chunk == mc, iota_c, big), axis=1, keepdims=True)
        win = mc < v
        idx = jnp.where(win, ic, idx)
        v = jnp.where(win, mc, v)
        u = lax.bitcast_convert_type(v, jnp.uint32)
        u = (u + jnp.uint32(0x8000)) & jnp.uint32(0xFFFF0000)
        v = lax.bitcast_convert_type(u, jnp.float32)
    idx_ref[...] = idx
    iota = lax.broadcasted_iota(jnp.int32, (TILE, K), 1)
    onehot = (iota == idx).astype(jnp.float32)
    counts_acc[...] += jnp.sum(onehot, axis=0, keepdims=True)

    @pl.when(i == GRID - 1)
    def _():
        counts_ref[...] = counts_acc[...]


@functools.partial(
    pl.kernel,
    mesh=plsc.VectorSubcoreMesh(core_axis_name="c", subcore_axis_name="s"),
    compiler_params=pltpu.CompilerParams(use_tc_tiling_on_sc=False),
    out_type=jax.ShapeDtypeStruct((N_ROWS, D), jnp.float32),
    scratch_types=[
        pltpu.VMEM((2, 128), jnp.int32),
        pltpu.VMEM((BPW, D), jnp.float32),
        pltpu.SemaphoreType.DMA,
    ],
)
def _sc_gather(table_hbm, idx_hbm, out_hbm, idx_v, rows_v, sem):
    wid = lax.axis_index("s") * 2 + lax.axis_index("c")
    pltpu.sync_copy(idx_hbm.at[pl.ds(wid * 2, 2)], idx_v)
    pltpu.async_copy(table_hbm.at[idx_v.at[0]], rows_v.at[pl.ds(0, 128)],
                     sem).wait()
    pltpu.async_copy(table_hbm.at[idx_v.at[1]], rows_v.at[pl.ds(128, 128)],
                     sem).wait()
    pltpu.sync_copy(rows_v, out_hbm.at[pl.ds(wid * BPW, BPW)])


def _finish_kernel(x_ref, q_ref, counts_ref, qst_ref, diff_ref, ppl_ref):
    x = x_ref[...]
    q = q_ref[...]
    d = q - x
    qst_ref[...] = x + d
    diff_ref[...] = (jnp.sum(d * d) / (N_ROWS * D)).reshape(1, 1)
    p = counts_ref[...] * (1.0 / N_ROWS)
    ent = jnp.sum(p * jnp.log(p + 1e-10))
    ppl_ref[...] = jnp.exp(-ent).reshape(1, 1)


def kernel(x, embed):
    x_flat = x.reshape(-1, D)
    x2 = (x_flat ** 2).sum(axis=1, keepdims=True)
    e2 = (embed ** 2).sum(axis=0, keepdims=True)
    embed_t = embed.T

    idx2, counts = pl.pallas_call(
        _vq_tile_kernel,
        grid=(GRID,),
        in_specs=[
            pl.BlockSpec((TILE, D), lambda i: (i, 0)),
            pl.BlockSpec((TILE, 1), lambda i: (i, 0)),
            pl.BlockSpec((1, K), lambda i: (0, 0)),
            pl.BlockSpec((D, K), lambda i: (0, 0)),
        ],
        out_specs=[
            pl.BlockSpec((TILE, 1), lambda i: (i, 0)),
            pl.BlockSpec((1, K), lambda i: (0, 0)),
        ],
        out_shape=[
            jax.ShapeDtypeStruct((N_ROWS, 1), jnp.int32),
            jax.ShapeDtypeStruct((1, K), jnp.float32),
        ],
        scratch_shapes=[pltpu.VMEM((1, K), jnp.float32)],
    )(x_flat, x2, e2, embed)

    quant = _sc_gather(embed_t, idx2.reshape(NW * 2, 128))

    qst, diff, ppl = pl.pallas_call(
        _finish_kernel,
        out_shape=[
            jax.ShapeDtypeStruct((N_ROWS, D), jnp.float32),
            jax.ShapeDtypeStruct((1, 1), jnp.float32),
            jax.ShapeDtypeStruct((1, 1), jnp.float32),
        ],
    )(x_flat, quant, counts)

    quantized_st = qst.reshape(x.shape)
    encoding_inds = idx2.reshape(x.shape[:-1])
    return (quantized_st, diff.reshape(()), encoding_inds, ppl.reshape(()))


# speedup vs baseline: 2.5357x; 1.2312x over previous
import functools

import jax
import jax.numpy as jnp
from jax import lax
from jax.experimental import pallas as pl
from jax.experimental.pallas import tpu as pltpu
from jax.experimental.pallas import tpu_sc as plsc

N_ROWS = 8192
K = 8192
D = 32
TILE = 256
GRID = N_ROWS // TILE
CHUNK = 2048

NW = 32
BPW = N_ROWS // NW


def _vq_tile_kernel(x_ref, x2_ref, e2_ref, embed_ref, idx_ref):
    x = x_ref[...]
    xe = jnp.dot(x, embed_ref[...], preferred_element_type=jnp.float32)
    scores = (x2_ref[...] - 2.0 * xe) + e2_ref[...]
    big = jnp.int32(2 ** 30)
    v = jnp.full((TILE, 1), jnp.inf, jnp.float32)
    idx = jnp.zeros((TILE, 1), jnp.int32)
    for c in range(K // CHUNK):
        chunk = scores[:, c * CHUNK:(c + 1) * CHUNK]
        mc = jnp.min(chunk, axis=1, keepdims=True)
        iota_c = lax.broadcasted_iota(jnp.int32, (TILE, CHUNK), 1) + c * CHUNK
        ic = jnp.min(jnp.where(chunk == mc, iota_c, big), axis=1, keepdims=True)
        win = mc < v
        idx = jnp.where(win, ic, idx)
        v = jnp.where(win, mc, v)
        u = lax.bitcast_convert_type(v, jnp.uint32)
        u = (u + jnp.uint32(0x8000)) & jnp.uint32(0xFFFF0000)
        v = lax.bitcast_convert_type(u, jnp.float32)
    idx_ref[...] = idx


@functools.partial(
    pl.kernel,
    mesh=plsc.VectorSubcoreMesh(core_axis_name="c", subcore_axis_name="s"),
    compiler_params=pltpu.CompilerParams(use_tc_tiling_on_sc=False,
                                         needs_layout_passes=False),
    out_type=[
        jax.ShapeDtypeStruct((N_ROWS, D), jnp.float32),
        jax.ShapeDtypeStruct((NW, K), jnp.float32),
    ],
    scratch_types=[
        pltpu.VMEM((2, 128), jnp.int32),
        pltpu.VMEM((BPW, D), jnp.float32),
        pltpu.VMEM((K,), jnp.float32),
        pltpu.SemaphoreType.DMA,
    ],
)
def _sc_gather_hist(table_hbm, idx_hbm, zeros_hbm, out_hbm, hist_hbm,
                    idx_v, rows_v, hist_v, sem):
    wid = lax.axis_index("s") * 2 + lax.axis_index("c")
    pltpu.sync_copy(idx_hbm.at[pl.ds(wid * 2, 2)], idx_v)
    pltpu.sync_copy(zeros_hbm, hist_v)
    pltpu.async_copy(table_hbm.at[idx_v.at[0]], rows_v.at[pl.ds(0, 128)],
                     sem).wait()
    pltpu.async_copy(table_hbm.at[idx_v.at[1]], rows_v.at[pl.ds(128, 128)],
                     sem).wait()
    ones = jnp.ones((16,), jnp.float32)
    for r in range(2):
        for j in range(128 // 16):
            iv = idx_v[r, pl.ds(j * 16, 16)]
            plsc.addupdate_scatter(hist_v, [iv], ones)
    pltpu.sync_copy(rows_v, out_hbm.at[pl.ds(wid * BPW, BPW)])
    pltpu.sync_copy(hist_v, hist_hbm.at[wid])


def _finish_kernel(x_ref, q_ref, hists_ref, qst_ref, diff_ref, ppl_ref):
    x = x_ref[...]
    q = q_ref[...]
    d = q - x
    qst_ref[...] = x + d
    diff_ref[...] = (jnp.sum(d * d) / (N_ROWS * D)).reshape(1, 1)
    counts = jnp.sum(hists_ref[...], axis=0, keepdims=True)
    p = counts * (1.0 / N_ROWS)
    ent = jnp.sum(p * jnp.log(p + 1e-10))
    ppl_ref[...] = jnp.exp(-ent).reshape(1, 1)


def kernel(x, embed):
    x_flat = x.reshape(-1, D)
    x2 = (x_flat ** 2).sum(axis=1, keepdims=True)
    e2 = (embed ** 2).sum(axis=0, keepdims=True)
    embed_t = embed.T
    zeros_k = jnp.zeros((K,), jnp.float32)

    idx2 = pl.pallas_call(
        _vq_tile_kernel,
        grid=(GRID,),
        in_specs=[
            pl.BlockSpec((TILE, D), lambda i: (i, 0)),
            pl.BlockSpec((TILE, 1), lambda i: (i, 0)),
            pl.BlockSpec((1, K), lambda i: (0, 0)),
            pl.BlockSpec((D, K), lambda i: (0, 0)),
        ],
        out_specs=pl.BlockSpec((TILE, 1), lambda i: (i, 0)),
        out_shape=jax.ShapeDtypeStruct((N_ROWS, 1), jnp.int32),
    )(x_flat, x2, e2, embed)

    quant, hists = _sc_gather_hist(embed_t, idx2.reshape(NW * 2, 128), zeros_k)

    qst, diff, ppl = pl.pallas_call(
        _finish_kernel,
        out_shape=[
            jax.ShapeDtypeStruct((N_ROWS, D), jnp.float32),
            jax.ShapeDtypeStruct((1, 1), jnp.float32),
            jax.ShapeDtypeStruct((1, 1), jnp.float32),
        ],
    )(x_flat, quant, hists)

    quantized_st = qst.reshape(x.shape)
    encoding_inds = idx2.reshape(x.shape[:-1])
    return (quantized_st, diff.reshape(()), encoding_inds, ppl.reshape(()))


# speedup vs baseline: 2.5369x; 1.0005x over previous
import functools

import jax
import jax.numpy as jnp
from jax import lax
from jax.experimental import pallas as pl
from jax.experimental.pallas import tpu as pltpu
from jax.experimental.pallas import tpu_sc as plsc

N_ROWS = 8192
K = 8192
D = 32
TILE = 256
GRID = N_ROWS // TILE
CHUNK = 2048

NW = 32
BPW = N_ROWS // NW


def _vq_tile_kernel(x_ref, x2_ref, e2_ref, embed_ref, idx_ref):
    x = x_ref[...]
    xe = jnp.dot(x, embed_ref[...], preferred_element_type=jnp.float32)
    scores = (x2_ref[...] - 2.0 * xe) + e2_ref[...]
    big = jnp.int32(2 ** 30)
    v = jnp.full((TILE, 1), jnp.inf, jnp.float32)
    idx = jnp.zeros((TILE, 1), jnp.int32)
    iota = lax.broadcasted_iota(jnp.int32, (TILE, K), 1)
    for c in range(K // CHUNK):
        chunk = scores[:, c * CHUNK:(c + 1) * CHUNK]
        mc = jnp.min(chunk, axis=1, keepdims=True)
        iota_c = iota[:, c * CHUNK:(c + 1) * CHUNK]
        ic = jnp.min(jnp.where(chunk == mc, iota_c, big), axis=1, keepdims=True)
        win = mc < v
        idx = jnp.where(win, ic, idx)
        v = jnp.where(win, mc, v)
        u = lax.bitcast_convert_type(v, jnp.uint32)
        u = (u + jnp.uint32(0x8000)) & jnp.uint32(0xFFFF0000)
        v = lax.bitcast_convert_type(u, jnp.float32)
    idx_ref[...] = idx


@functools.partial(
    pl.kernel,
    mesh=plsc.VectorSubcoreMesh(core_axis_name="c", subcore_axis_name="s"),
    compiler_params=pltpu.CompilerParams(use_tc_tiling_on_sc=False,
                                         needs_layout_passes=False),
    out_type=[
        jax.ShapeDtypeStruct((N_ROWS, D), jnp.float32),
        jax.ShapeDtypeStruct((NW, K), jnp.float32),
    ],
    scratch_types=[
        pltpu.VMEM((2, 128), jnp.int32),
        pltpu.VMEM((BPW, D), jnp.float32),
        pltpu.VMEM((K,), jnp.float32),
        pltpu.SemaphoreType.DMA,
    ],
)
def _sc_gather_hist(table_hbm, idx_hbm, zeros_hbm, out_hbm, hist_hbm,
                    idx_v, rows_v, hist_v, sem):
    wid = lax.axis_index("s") * 2 + lax.axis_index("c")
    pltpu.sync_copy(idx_hbm.at[pl.ds(wid * 2, 2)], idx_v)
    pltpu.sync_copy(zeros_hbm, hist_v)
    pltpu.async_copy(table_hbm.at[idx_v.at[0]], rows_v.at[pl.ds(0, 128)],
                     sem).wait()
    pltpu.async_copy(table_hbm.at[idx_v.at[1]], rows_v.at[pl.ds(128, 128)],
                     sem).wait()
    ones = jnp.ones((16,), jnp.float32)
    for r in range(2):
        for j in range(128 // 16):
            iv = idx_v[r, pl.ds(j * 16, 16)]
            plsc.addupdate_scatter(hist_v, [iv], ones)
    pltpu.sync_copy(rows_v, out_hbm.at[pl.ds(wid * BPW, BPW)])
    pltpu.sync_copy(hist_v, hist_hbm.at[wid])


def _finish_kernel(x_ref, q_ref, hists_ref, qst_ref, diff_ref, ppl_ref):
    x = x_ref[...]
    q = q_ref[...]
    d = q - x
    qst_ref[...] = x + d
    diff_ref[...] = (jnp.sum(d * d) / (N_ROWS * D)).reshape(1, 1)
    counts = jnp.sum(hists_ref[...], axis=0, keepdims=True)
    p = counts * (1.0 / N_ROWS)
    ent = jnp.sum(p * jnp.log(p + 1e-10))
    ppl_ref[...] = jnp.exp(-ent).reshape(1, 1)


def kernel(x, embed):
    x_flat = x.reshape(-1, D)
    x2 = (x_flat ** 2).sum(axis=1, keepdims=True)
    e2 = (embed ** 2).sum(axis=0, keepdims=True)
    embed_t = embed.T
    zeros_k = jnp.zeros((K,), jnp.float32)

    idx2 = pl.pallas_call(
        _vq_tile_kernel,
        grid=(GRID,),
        in_specs=[
            pl.BlockSpec((TILE, D), lambda i: (i, 0)),
            pl.BlockSpec((TILE, 1), lambda i: (i, 0)),
            pl.BlockSpec((1, K), lambda i: (0, 0)),
            pl.BlockSpec((D, K), lambda i: (0, 0)),
        ],
        out_specs=pl.BlockSpec((TILE, 1), lambda i: (i, 0)),
        out_shape=jax.ShapeDtypeStruct((N_ROWS, 1), jnp.int32),
    )(x_flat, x2, e2, embed)

    quant, hists = _sc_gather_hist(embed_t, idx2.reshape(NW * 2, 128), zeros_k)

    qst, diff, ppl = pl.pallas_call(
        _finish_kernel,
        out_shape=[
            jax.ShapeDtypeStruct((N_ROWS, D), jnp.float32),
            jax.ShapeDtypeStruct((1, 1), jnp.float32),
            jax.ShapeDtypeStruct((1, 1), jnp.float32),
        ],
    )(x_flat, quant, hists)

    quantized_st = qst.reshape(x.shape)
    encoding_inds = idx2.reshape(x.shape[:-1])
    return (quantized_st, diff.reshape(()), encoding_inds, ppl.reshape(()))


# speedup vs baseline: 2.6834x; 1.0577x over previous
import functools

import jax
import jax.numpy as jnp
from jax import lax
from jax.experimental import pallas as pl
from jax.experimental.pallas import tpu as pltpu
from jax.experimental.pallas import tpu_sc as plsc

N_ROWS = 8192
K = 8192
D = 32
TILE = 512
GRID = N_ROWS // TILE
CHUNK = 2048

NW = 32
BPW = N_ROWS // NW


def _vq_tile_kernel(x_ref, x2_ref, e2_ref, embed_ref, idx_ref):
    x = x_ref[...]
    xe = jnp.dot(x, embed_ref[...], preferred_element_type=jnp.float32)
    scores = (x2_ref[...] - 2.0 * xe) + e2_ref[...]
    big = jnp.int32(2 ** 30)
    v = jnp.full((TILE, 1), jnp.inf, jnp.float32)
    idx = jnp.zeros((TILE, 1), jnp.int32)
    iota = lax.broadcasted_iota(jnp.int32, (TILE, K), 1)
    for c in range(K // CHUNK):
        chunk = scores[:, c * CHUNK:(c + 1) * CHUNK]
        mc = jnp.min(chunk, axis=1, keepdims=True)
        iota_c = iota[:, c * CHUNK:(c + 1) * CHUNK]
        ic = jnp.min(jnp.where(chunk == mc, iota_c, big), axis=1, keepdims=True)
        win = mc < v
        idx = jnp.where(win, ic, idx)
        v = jnp.where(win, mc, v)
        u = lax.bitcast_convert_type(v, jnp.uint32)
        u = (u + jnp.uint32(0x8000)) & jnp.uint32(0xFFFF0000)
        v = lax.bitcast_convert_type(u, jnp.float32)
    idx_ref[...] = idx


@functools.partial(
    pl.kernel,
    mesh=plsc.VectorSubcoreMesh(core_axis_name="c", subcore_axis_name="s"),
    compiler_params=pltpu.CompilerParams(use_tc_tiling_on_sc=False,
                                         needs_layout_passes=False),
    out_type=[
        jax.ShapeDtypeStruct((N_ROWS, D), jnp.float32),
        jax.ShapeDtypeStruct((NW, K), jnp.float32),
    ],
    scratch_types=[
        pltpu.VMEM((2, 128), jnp.int32),
        pltpu.VMEM((BPW, D), jnp.float32),
        pltpu.VMEM((K,), jnp.float32),
        pltpu.SemaphoreType.DMA,
    ],
)
def _sc_gather_hist(table_hbm, idx_hbm, zeros_hbm, out_hbm, hist_hbm,
                    idx_v, rows_v, hist_v, sem):
    wid = lax.axis_index("s") * 2 + lax.axis_index("c")
    pltpu.sync_copy(idx_hbm.at[pl.ds(wid * 2, 2)], idx_v)
    pltpu.sync_copy(zeros_hbm, hist_v)
    pltpu.async_copy(table_hbm.at[idx_v.at[0]], rows_v.at[pl.ds(0, 128)],
                     sem).wait()
    pltpu.async_copy(table_hbm.at[idx_v.at[1]], rows_v.at[pl.ds(128, 128)],
                     sem).wait()
    ones = jnp.ones((16,), jnp.float32)
    for r in range(2):
        for j in range(128 // 16):
            iv = idx_v[r, pl.ds(j * 16, 16)]
            plsc.addupdate_scatter(hist_v, [iv], ones)
    pltpu.sync_copy(rows_v, out_hbm.at[pl.ds(wid * BPW, BPW)])
    pltpu.sync_copy(hist_v, hist_hbm.at[wid])


def _finish_kernel(x_ref, q_ref, hists_ref, qst_ref, diff_ref, ppl_ref):
    x = x_ref[...]
    q = q_ref[...]
    d = q - x
    qst_ref[...] = x + d
    diff_ref[...] = (jnp.sum(d * d) / (N_ROWS * D)).reshape(1, 1)
    counts = jnp.sum(hists_ref[...], axis=0, keepdims=True)
    p = counts * (1.0 / N_ROWS)
    ent = jnp.sum(p * jnp.log(p + 1e-10))
    ppl_ref[...] = jnp.exp(-ent).reshape(1, 1)


def kernel(x, embed):
    x_flat = x.reshape(-1, D)
    x2 = (x_flat ** 2).sum(axis=1, keepdims=True)
    e2 = (embed ** 2).sum(axis=0, keepdims=True)
    embed_t = embed.T
    zeros_k = jnp.zeros((K,), jnp.float32)

    idx2 = pl.pallas_call(
        _vq_tile_kernel,
        grid=(GRID,),
        in_specs=[
            pl.BlockSpec((TILE, D), lambda i: (i, 0)),
            pl.BlockSpec((TILE, 1), lambda i: (i, 0)),
            pl.BlockSpec((1, K), lambda i: (0, 0)),
            pl.BlockSpec((D, K), lambda i: (0, 0)),
        ],
        out_specs=pl.BlockSpec((TILE, 1), lambda i: (i, 0)),
        out_shape=jax.ShapeDtypeStruct((N_ROWS, 1), jnp.int32),
    )(x_flat, x2, e2, embed)

    quant, hists = _sc_gather_hist(embed_t, idx2.reshape(NW * 2, 128), zeros_k)

    qst, diff, ppl = pl.pallas_call(
        _finish_kernel,
        out_shape=[
            jax.ShapeDtypeStruct((N_ROWS, D), jnp.float32),
            jax.ShapeDtypeStruct((1, 1), jnp.float32),
            jax.ShapeDtypeStruct((1, 1), jnp.float32),
        ],
    )(x_flat, quant, hists)

    quantized_st = qst.reshape(x.shape)
    encoding_inds = idx2.reshape(x.shape[:-1])
    return (quantized_st, diff.reshape(()), encoding_inds, ppl.reshape(()))
